# Initial kernel scaffold; baseline (speedup 1.0000x reference)
import jax
import jax.numpy as jnp
from jax.experimental import pallas as pl


def kernel(input, embd_weights):
    raise NotImplementedError("write your pallas kernel here")


# speedup vs baseline: 2.7676x; 2.7676x over previous
import functools

import jax
import jax.numpy as jnp
from jax import lax
from jax.experimental import pallas as pl
from jax.experimental.pallas import tpu as pltpu, tpu_sc as plsc

BATCH = 4096
SEQ = 200
DIM = 64
NC, NS, L = 2, 16, 16
NW = NC * NS
ROWS_PER_W = BATCH // NW
TOK_PER_W = ROWS_PER_W * SEQ
GROUP_ROWS = 16
GROUP_TOK = GROUP_ROWS * SEQ
N_GROUPS = ROWS_PER_W // GROUP_ROWS
GATHER_ROWS = 128
N_GATHER = GROUP_TOK // GATHER_ROWS


def _pos_embed_sc(tok_hbm, table_hbm, out_hbm, toks_v, idx_v, rows_v, sem):
    wid = lax.axis_index("s") * NC + lax.axis_index("c")
    base = wid * TOK_PER_W
    rowoff = lax.iota(jnp.int32, L) * SEQ
    ones = jnp.ones((L,), jnp.int32)
    zeros = jnp.zeros((L,), jnp.int32)

    def group_body(g, _):
        gbase = base + g * GROUP_TOK
        pltpu.sync_copy(tok_hbm.at[pl.ds(gbase, GROUP_TOK)], toks_v)

        def pos_body(s, carry):
            idx = rowoff + s
            tok = plsc.load_gather(toks_v, [idx])
            m = jnp.where(tok != 0, ones, zeros)
            carry = carry + m
            plsc.store_scatter(idx_v, [idx], carry * m)
            return carry

        lax.fori_loop(0, SEQ, pos_body, zeros)

        def gather_body(j, _):
            o = j * GATHER_ROWS
            pltpu.async_copy(
                table_hbm.at[idx_v.at[pl.ds(o, GATHER_ROWS)]], rows_v, sem
            ).wait()
            pltpu.sync_copy(rows_v, out_hbm.at[pl.ds(gbase + o, GATHER_ROWS)])
            return 0

        lax.fori_loop(0, N_GATHER, gather_body, 0)
        return 0

    lax.fori_loop(0, N_GROUPS, group_body, 0)


@jax.jit
def kernel(input, embd_weights):
    tok_flat = input.reshape(-1).astype(jnp.int32)
    mesh = plsc.VectorSubcoreMesh(core_axis_name="c", subcore_axis_name="s")
    out = pl.kernel(
        _pos_embed_sc,
        out_type=jax.ShapeDtypeStruct((BATCH * SEQ, DIM), jnp.float32),
        mesh=mesh,
        scratch_types=[
            pltpu.VMEM((GROUP_TOK,), jnp.int32),
            pltpu.VMEM((GROUP_TOK,), jnp.int32),
            pltpu.VMEM((GATHER_ROWS, DIM), jnp.float32),
            pltpu.SemaphoreType.DMA,
        ],
        compiler_params=pltpu.CompilerParams(
            needs_layout_passes=False, use_tc_tiling_on_sc=False
        ),
    )(tok_flat, embd_weights)
    return out.reshape(BATCH, SEQ, DIM)


# speedup vs baseline: 2.7959x; 1.0102x over previous
import jax
import jax.numpy as jnp
from jax import lax
from jax.experimental import pallas as pl
from jax.experimental.pallas import tpu as pltpu, tpu_sc as plsc

BATCH = 4096
SEQ = 200
DIM = 64
NC, NS, L = 2, 16, 16
NW = NC * NS
ROWS_PER_W = BATCH // NW
TOK_PER_W = ROWS_PER_W * SEQ
GROUP_ROWS = 16
GROUP_TOK = GROUP_ROWS * SEQ
N_GROUPS = ROWS_PER_W // GROUP_ROWS
GATHER_ROWS = 128
PHASE_GATHERS = 5
PHASE_ROWS = GATHER_ROWS * PHASE_GATHERS
PHASES_PER_GROUP = GROUP_TOK // PHASE_ROWS
N_PHASES = N_GROUPS * PHASES_PER_GROUP


def _pos_embed_sc(tok_hbm, table_hbm, out_hbm, toks_v, idx_v, rows_v,
                  tsem, gsem, wsem):
    wid = lax.axis_index("s") * NC + lax.axis_index("c")
    base = wid * TOK_PER_W
    rowoff = lax.iota(jnp.int32, L) * SEQ
    ones = jnp.ones((L,), jnp.int32)
    zeros = jnp.zeros((L,), jnp.int32)

    def fire_tok(g):
        return pltpu.async_copy(
            tok_hbm.at[pl.ds(base + g * GROUP_TOK, GROUP_TOK)],
            toks_v.at[g % 2], tsem)

    def compute_positions(g):
        tv = toks_v.at[g % 2]
        iv = idx_v.at[g % 2]

        def pos_body(s, carry):
            idx = rowoff + s
            tok = plsc.load_gather(tv, [idx])
            m = jnp.where(tok != 0, ones, zeros)
            carry = carry + m
            plsc.store_scatter(iv, [idx], carry * m)
            return carry

        lax.fori_loop(0, SEQ, pos_body, zeros)

    def fire_gathers(ph):
        g, p = divmod(ph, PHASES_PER_GROUP)
        h = ph % 2
        descs = []
        for i in range(PHASE_GATHERS):
            o = (p * PHASE_GATHERS + i) * GATHER_ROWS
            descs.append(pltpu.async_copy(
                table_hbm.at[idx_v.at[g % 2, pl.ds(o, GATHER_ROWS)]],
                rows_v.at[h, pl.ds(i * GATHER_ROWS, GATHER_ROWS)], gsem))
        return descs

    def fire_write(ph):
        g, p = divmod(ph, PHASES_PER_GROUP)
        h = ph % 2
        return pltpu.async_copy(
            rows_v.at[h],
            out_hbm.at[pl.ds(base + g * GROUP_TOK + p * PHASE_ROWS,
                             PHASE_ROWS)], wsem)

    gdescs, wdescs = {}, {}
    tok_desc = fire_tok(0)
    for g in range(N_GROUPS):
        tok_desc.wait()
        compute_positions(g)
        if g + 1 < N_GROUPS:
            tok_desc = fire_tok(g + 1)
        for p in range(PHASES_PER_GROUP):
            ph = g * PHASES_PER_GROUP + p
            if ph >= 1:
                for d in gdescs.pop(ph - 1):
                    d.wait()
                wdescs[ph - 1] = fire_write(ph - 1)
            if ph >= 2:
                wdescs.pop(ph - 2).wait()
            gdescs[ph] = fire_gathers(ph)
    last = N_PHASES - 1
    for d in gdescs.pop(last):
        d.wait()
    wdescs[last] = fire_write(last)
    wdescs.pop(last - 1).wait()
    wdescs.pop(last).wait()


@jax.jit
def kernel(input, embd_weights):
    tok_flat = input.reshape(-1).astype(jnp.int32)
    mesh = plsc.VectorSubcoreMesh(core_axis_name="c", subcore_axis_name="s")
    out = pl.kernel(
        _pos_embed_sc,
        out_type=jax.ShapeDtypeStruct((BATCH * SEQ, DIM), jnp.float32),
        mesh=mesh,
        scratch_types=[
            pltpu.VMEM((2, GROUP_TOK), jnp.int32),
            pltpu.VMEM((2, GROUP_TOK), jnp.int32),
            pltpu.VMEM((2, PHASE_ROWS, DIM), jnp.float32),
            pltpu.SemaphoreType.DMA,
            pltpu.SemaphoreType.DMA,
            pltpu.SemaphoreType.DMA,
        ],
        compiler_params=pltpu.CompilerParams(
            needs_layout_passes=False, use_tc_tiling_on_sc=False
        ),
    )(tok_flat, embd_weights)
    return out.reshape(BATCH, SEQ, DIM)


# speedup vs baseline: 5.0202x; 1.7955x over previous
import jax
import jax.numpy as jnp
from jax import lax
from jax.experimental import pallas as pl
from jax.experimental.pallas import tpu as pltpu, tpu_sc as plsc

BATCH = 4096
SEQ = 200
DIM = 64
NC, NS, L = 2, 16, 16
NW = NC * NS
ROWS_PER_W = BATCH // NW
TOK_PER_W = ROWS_PER_W * SEQ
GROUP_ROWS = 16
GROUP_TOK = GROUP_ROWS * SEQ
N_GROUPS = ROWS_PER_W // GROUP_ROWS
GATHER_ROWS = 128
PHASE_GATHERS = 5
PHASE_ROWS = GATHER_ROWS * PHASE_GATHERS
PHASES_PER_GROUP = GROUP_TOK // PHASE_ROWS
N_PHASES = N_GROUPS * PHASES_PER_GROUP


TABLE_ROWS = 208


def _pos_embed_sc(tok_hbm, table_hbm, out_hbm, toks_v, idx_v, rows_v,
                  table_sp, tsem, gsem, wsem):
    wid = lax.axis_index("s") * NC + lax.axis_index("c")
    base = wid * TOK_PER_W

    @pl.when(lax.axis_index("s") == 0)
    def _():
        pltpu.sync_copy(table_hbm.at[pl.ds(0, TABLE_ROWS)], table_sp)

    plsc.subcore_barrier()
    rowoff = lax.iota(jnp.int32, L) * SEQ
    ones = jnp.ones((L,), jnp.int32)
    zeros = jnp.zeros((L,), jnp.int32)

    def fire_tok(g):
        return pltpu.async_copy(
            tok_hbm.at[pl.ds(base + g * GROUP_TOK, GROUP_TOK)],
            toks_v.at[g % 2], tsem)

    def compute_positions(g):
        tv = toks_v.at[g % 2]
        iv = idx_v.at[g % 2]

        def pos_body(s, carry):
            idx = rowoff + s
            tok = plsc.load_gather(tv, [idx])
            m = jnp.where(tok != 0, ones, zeros)
            carry = carry + m
            plsc.store_scatter(iv, [idx], carry * m)
            return carry

        lax.fori_loop(0, SEQ, pos_body, zeros)

    def fire_gathers(ph):
        g, p = divmod(ph, PHASES_PER_GROUP)
        h = ph % 2
        descs = []
        for i in range(PHASE_GATHERS):
            o = (p * PHASE_GATHERS + i) * GATHER_ROWS
            descs.append(pltpu.async_copy(
                table_sp.at[idx_v.at[g % 2, pl.ds(o, GATHER_ROWS)]],
                rows_v.at[h, pl.ds(i * GATHER_ROWS, GATHER_ROWS)], gsem))
        return descs

    def fire_write(ph):
        g, p = divmod(ph, PHASES_PER_GROUP)
        h = ph % 2
        return pltpu.async_copy(
            rows_v.at[h],
            out_hbm.at[pl.ds(base + g * GROUP_TOK + p * PHASE_ROWS,
                             PHASE_ROWS)], wsem)

    gdescs, wdescs = {}, {}
    tok_desc = fire_tok(0)
    for g in range(N_GROUPS):
        tok_desc.wait()
        compute_positions(g)
        if g + 1 < N_GROUPS:
            tok_desc = fire_tok(g + 1)
        for p in range(PHASES_PER_GROUP):
            ph = g * PHASES_PER_GROUP + p
            if ph >= 1:
                for d in gdescs.pop(ph - 1):
                    d.wait()
                wdescs[ph - 1] = fire_write(ph - 1)
            if ph >= 2:
                wdescs.pop(ph - 2).wait()
            gdescs[ph] = fire_gathers(ph)
    last = N_PHASES - 1
    for d in gdescs.pop(last):
        d.wait()
    wdescs[last] = fire_write(last)
    wdescs.pop(last - 1).wait()
    wdescs.pop(last).wait()


@jax.jit
def kernel(input, embd_weights):
    tok_flat = input.reshape(-1).astype(jnp.int32)
    mesh = plsc.VectorSubcoreMesh(core_axis_name="c", subcore_axis_name="s")
    out = pl.kernel(
        _pos_embed_sc,
        out_type=jax.ShapeDtypeStruct((BATCH * SEQ, DIM), jnp.float32),
        mesh=mesh,
        scratch_types=[
            pltpu.VMEM((2, GROUP_TOK), jnp.int32),
            pltpu.VMEM((2, GROUP_TOK), jnp.int32),
            pltpu.VMEM((2, PHASE_ROWS, DIM), jnp.float32),
            pltpu.VMEM_SHARED((TABLE_ROWS, DIM), jnp.float32),
            pltpu.SemaphoreType.DMA,
            pltpu.SemaphoreType.DMA,
            pltpu.SemaphoreType.DMA,
        ],
        compiler_params=pltpu.CompilerParams(
            needs_layout_passes=False, use_tc_tiling_on_sc=False
        ),
    )(tok_flat, embd_weights)
    return out.reshape(BATCH, SEQ, DIM)
